# baseline (speedup 1.0000x reference)
import functools

import jax
import jax.numpy as jnp
from jax import lax
from jax.experimental import pallas as pl
from jax.experimental.pallas import tpu as pltpu
from jax.experimental.pallas import tpu_sc as plsc

BS = 8
C = 8
N = 262144
L = 5
NC = 2
NS = 16
NW = NC * NS
PW = N // NW
CHUNK = 2048
NCHUNK = PW // CHUNK
CHSHIFT = NCHUNK.bit_length() - 1
NVEC = CHUNK // 16
NFLAT = BS * NCHUNK
DELTA_V = 0.5
DELTA_D = 3.0
ROWB = (C + 1) * 16

_mesh = plsc.VectorSubcoreMesh(core_axis_name="core", subcore_axis_name="sub")
_params = pltpu.CompilerParams(needs_layout_passes=False)
_PAIRS = [(1, 2), (1, 3), (1, 4), (2, 3), (2, 4), (3, 4)]


def _wid():
    return lax.axis_index("sub") * NC + lax.axis_index("core")


def _vsum(v):
    return jnp.sum(v)


def _srecip(x):
    return (1.0 / jnp.full((16,), x, jnp.float32))[0]


def _vrsqrt(x):
    i = plsc.bitcast(x, jnp.int32)
    i = jnp.int32(0x5F3759DF) - lax.shift_right_logical(i, 1)
    y = plsc.bitcast(i, jnp.float32)
    xh = 0.5 * x
    for _ in range(3):
        y = y * (1.5 - xh * y * y)
    return y


def _issue(emb_hbm, seg_hbm, emb_v, seg_v, sem, base, nk):
    b = lax.shift_right_logical(nk, CHSHIFT)
    ch = nk & (NCHUNK - 1)
    start = base + ch * CHUNK
    par = nk & 3
    pltpu.async_copy(emb_hbm.at[b, :, pl.ds(start, CHUNK)], emb_v.at[par],
                     sem.at[par])
    pltpu.async_copy(seg_hbm.at[b, pl.ds(start, CHUNK)], seg_v.at[par],
                     sem.at[par])


def _drain(emb_hbm, seg_hbm, emb_v, seg_v, sem, par):
    pltpu.make_async_copy(emb_hbm.at[0, :, pl.ds(0, CHUNK)], emb_v.at[par],
                          sem.at[par]).wait()
    pltpu.make_async_copy(seg_hbm.at[0, pl.ds(0, CHUNK)], seg_v.at[par],
                          sem.at[par]).wait()


@functools.partial(
    pl.kernel,
    out_type=jax.ShapeDtypeStruct((NW, BS * ROWB), jnp.float32),
    mesh=_mesh,
    compiler_params=_params,
    scratch_types=[
        pltpu.VMEM((4, C, CHUNK), jnp.float32),
        pltpu.VMEM((4, CHUNK), jnp.int32),
        pltpu.VMEM((BS * ROWB,), jnp.float32),
        pltpu.SemaphoreType.DMA((4,)),
    ],
)
def _moments(emb_hbm, seg_hbm, out_hbm, emb_v, seg_v, row_v, sem):
    base = _wid() * PW
    zvec = jnp.zeros((16,), jnp.float32)
    iota = lax.iota(jnp.int32, 16)

    for pk in range(3):
        _issue(emb_hbm, seg_hbm, emb_v, seg_v, sem, base, pk)

    def flat_body(kk, accs):
        par = kk & 3
        nk = kk + 3

        @pl.when(nk < NFLAT)
        def _():
            _issue(emb_hbm, seg_hbm, emb_v, seg_v, sem, base, nk)

        _drain(emb_hbm, seg_hbm, emb_v, seg_v, sem, par)
        b = lax.shift_right_logical(kk, CHSHIFT)
        ch = kk & (NCHUNK - 1)

        def vec_body(i, accs):
            accs = list(accs)
            off = i * 16
            seg = seg_v[par, pl.ds(off, 16)]
            fs = [jnp.where(seg == l, 1.0, 0.0) for l in range(1, L)]
            for c in range(C):
                x = emb_v[par, c, pl.ds(off, 16)]
                for li in range(4):
                    accs[li * C + c] = accs[li * C + c] + fs[li] * x
            for li in range(4):
                accs[32 + li] = accs[32 + li] + fs[li]
            return tuple(accs)

        accs = lax.fori_loop(0, NVEC, vec_body, accs, unroll=2)
        last = ch == (NCHUNK - 1)

        @pl.when(last)
        def _():
            boffs = b * ROWB
            for c in range(C):
                row = zvec
                for li in range(4):
                    s = _vsum(accs[li * C + c])
                    row = jnp.where(iota == li + 1, s, row)
                row_v[pl.ds(boffs + c * 16, 16)] = row
            crow = zvec
            for li in range(4):
                s = _vsum(accs[32 + li])
                crow = jnp.where(iota == li + 1, s, crow)
            row_v[pl.ds(boffs + C * 16, 16)] = crow

        return tuple(jnp.where(last, zvec, a) for a in accs)

    lax.fori_loop(0, NFLAT, flat_body, (zvec,) * 36)
    pltpu.sync_copy(row_v, out_hbm.at[_wid()])


@functools.partial(
    pl.kernel,
    out_type=jax.ShapeDtypeStruct((NW, 16), jnp.float32),
    mesh=_mesh,
    compiler_params=_params,
    scratch_types=[
        pltpu.VMEM((4, C, CHUNK), jnp.float32),
        pltpu.VMEM((4, CHUNK), jnp.int32),
        pltpu.VMEM((NW, BS * ROWB), jnp.float32),
        pltpu.VMEM((BS, C, 16), jnp.float32),
        pltpu.VMEM((BS, 16), jnp.float32),
        pltpu.VMEM((16,), jnp.float32),
        pltpu.SemaphoreType.DMA((4,)),
    ],
)
def _hinge(emb_hbm, seg_hbm, part_hbm, out_hbm, emb_v, seg_v, part_v,
           mu_v, coef_v, row_v, sem):
    wid = _wid()
    base = wid * PW
    zvec = jnp.zeros((16,), jnp.float32)
    iota = lax.iota(jnp.int32, 16)
    cvecs = [jnp.full((16,), c, jnp.int32) for c in range(C)]
    lmask = (iota >= 1) & (iota < L)

    pltpu.sync_copy(part_hbm, part_v)

    dsq_rows = [zvec, zvec, zvec]
    w_rows = [zvec, zvec, zvec]
    for b in range(BS):
        def red_body(w, vs):
            vs = list(vs)
            for c in range(C + 1):
                vs[c] = vs[c] + part_v[w, pl.ds(b * ROWB + c * 16, 16)]
            return tuple(vs)

        red = lax.fori_loop(0, NW, red_body, (zvec,) * (C + 1))
        vcnt = red[C]
        invv = 1.0 / jnp.maximum(vcnt, 1.0)
        presf = jnp.where(lmask & (vcnt > 0.0), 1.0, 0.0)
        nl = _vsum(presf)
        inv_nl = _srecip(jnp.maximum(nl, 1.0))
        coef_v[b, pl.ds(0, 16)] = presf * invv * inv_nl * (1.0 / BS)
        mus = [red[c] * invv for c in range(C)]
        for c in range(C):
            mu_v[b, c, pl.ds(0, 16)] = mus[c]
        guard = jnp.where(nl > 1.0, 1.0, 0.0)
        invd = guard * _srecip(jnp.maximum(nl * (nl - 1.0), 1.0)) * (1.0 / BS)
        for p, (i, j) in enumerate(_PAIRS):
            dsq = jnp.float32(0.0)
            for c in range(C):
                d = mus[c][i] - mus[c][j]
                dsq = dsq + d * d
            e = b * 6 + p
            dsq_rows[e // 16] = jnp.where(iota == e % 16, dsq, dsq_rows[e // 16])
            wv = presf[i] * presf[j] * invd
            w_rows[e // 16] = jnp.where(iota == e % 16, wv, w_rows[e // 16])

    dist = jnp.float32(0.0)
    for k in range(3):
        dsq = jnp.maximum(dsq_rows[k], 1e-24)
        pd = dsq * _vrsqrt(dsq)
        h = jnp.maximum(DELTA_D - pd, 0.0)
        dist = dist + _vsum(w_rows[k] * h * h)

    for pk in range(3):
        _issue(emb_hbm, seg_hbm, emb_v, seg_v, sem, base, pk)

    def flat_body(kk, vacc):
        par = kk & 3
        nk = kk + 3

        @pl.when(nk < NFLAT)
        def _():
            _issue(emb_hbm, seg_hbm, emb_v, seg_v, sem, base, nk)

        _drain(emb_hbm, seg_hbm, emb_v, seg_v, sem, par)
        b = lax.shift_right_logical(kk, CHSHIFT)
        bvec = jnp.full((16,), 0, jnp.int32) + b

        def vec_body(i, vacc):
            off = i * 16
            seg = seg_v[par, pl.ds(off, 16)]
            sq = jnp.zeros((16,), jnp.float32)
            for c in range(C):
                x = emb_v[par, c, pl.ds(off, 16)]
                mu = plsc.load_gather(mu_v, [bvec, cvecs[c], seg])
                d = x - mu
                sq = sq + d * d
            norm = sq * _vrsqrt(sq)
            t = jnp.maximum(norm - DELTA_V, 0.0)
            cpx = plsc.load_gather(coef_v, [bvec, seg])
            return vacc + cpx * (t * t)

        return lax.fori_loop(0, NVEC, vec_body, vacc, unroll=2)

    vacc = lax.fori_loop(0, NFLAT, flat_body, zvec)
    var_w = _vsum(vacc)

    row = (jnp.where(iota == 0, var_w, 0.0)
           + jnp.where(iota == 1, dist, 0.0)).astype(jnp.float32)
    row_v[...] = row
    pltpu.sync_copy(row_v, out_hbm.at[wid])


def kernel(embedding, seg_gt):
    partials = _moments(embedding, seg_gt)
    out = _hinge(embedding, seg_gt, partials)
    var_loss = jnp.sum(out[:, 0])
    dist_loss = out[0, 1]
    return (var_loss, dist_loss, jnp.zeros((), jnp.float32))


# speedup vs baseline: 1.2448x; 1.2448x over previous
import functools

import jax
import jax.numpy as jnp
from jax import lax
from jax.experimental import pallas as pl
from jax.experimental.pallas import tpu as pltpu
from jax.experimental.pallas import tpu_sc as plsc

BS = 8
C = 8
N = 262144
L = 5
NC = 2
NS = 16
NW = NC * NS
PW = N // NW
CHUNK = 2048
NCHUNK = PW // CHUNK
CHSHIFT = NCHUNK.bit_length() - 1
NVEC = CHUNK // 16
NFLAT = BS * NCHUNK
DELTA_V = 0.5
DELTA_D = 3.0
ROWB = (C + 1) * 16

_mesh = plsc.VectorSubcoreMesh(core_axis_name="core", subcore_axis_name="sub")
_params = pltpu.CompilerParams(needs_layout_passes=False)
_PAIRS = [(1, 2), (1, 3), (1, 4), (2, 3), (2, 4), (3, 4)]


def _wid():
    return lax.axis_index("sub") * NC + lax.axis_index("core")


def _vsum(v):
    return jnp.sum(v)


def _srecip(x):
    return (1.0 / jnp.full((16,), x, jnp.float32))[0]


def _vrsqrt(x):
    i = plsc.bitcast(x, jnp.int32)
    i = jnp.int32(0x5F3759DF) - lax.shift_right_logical(i, 1)
    y = plsc.bitcast(i, jnp.float32)
    xh = 0.5 * x
    for _ in range(3):
        y = y * (1.5 - xh * y * y)
    return y


def _issue(emb_hbm, seg_hbm, emb_v, seg_v, sem, base, nk):
    b = lax.shift_right_logical(nk, CHSHIFT)
    ch = nk & (NCHUNK - 1)
    start = base + ch * CHUNK
    par = nk & 3
    pltpu.async_copy(emb_hbm.at[b, :, pl.ds(start, CHUNK)], emb_v.at[par],
                     sem.at[par])
    pltpu.async_copy(seg_hbm.at[b, pl.ds(start, CHUNK)], seg_v.at[par],
                     sem.at[par])


def _drain(emb_hbm, seg_hbm, emb_v, seg_v, sem, par):
    pltpu.make_async_copy(emb_hbm.at[0, :, pl.ds(0, CHUNK)], emb_v.at[par],
                          sem.at[par]).wait()
    pltpu.make_async_copy(seg_hbm.at[0, pl.ds(0, CHUNK)], seg_v.at[par],
                          sem.at[par]).wait()


@functools.partial(
    pl.kernel,
    out_type=jax.ShapeDtypeStruct((NW, BS * ROWB), jnp.float32),
    mesh=_mesh,
    compiler_params=_params,
    scratch_types=[
        pltpu.VMEM((4, C, CHUNK), jnp.float32),
        pltpu.VMEM((4, CHUNK), jnp.int32),
        pltpu.VMEM((BS * ROWB,), jnp.float32),
        pltpu.SemaphoreType.DMA((4,)),
    ],
)
def _moments(emb_hbm, seg_hbm, out_hbm, emb_v, seg_v, row_v, sem):
    base = _wid() * PW
    zvec = jnp.zeros((16,), jnp.float32)
    iota = lax.iota(jnp.int32, 16)

    for pk in range(3):
        _issue(emb_hbm, seg_hbm, emb_v, seg_v, sem, base, pk)

    def flat_body(kk, accs):
        par = kk & 3
        nk = kk + 3

        @pl.when(nk < NFLAT)
        def _():
            _issue(emb_hbm, seg_hbm, emb_v, seg_v, sem, base, nk)

        _drain(emb_hbm, seg_hbm, emb_v, seg_v, sem, par)
        b = lax.shift_right_logical(kk, CHSHIFT)
        ch = kk & (NCHUNK - 1)

        def vec_body(i, accs):
            accs = list(accs)
            off = i * 16
            seg = seg_v[par, pl.ds(off, 16)]
            fs = [jnp.where(seg == l, 1.0, 0.0) for l in range(1, L)]
            for c in range(C):
                x = emb_v[par, c, pl.ds(off, 16)]
                for li in range(4):
                    accs[li * C + c] = accs[li * C + c] + fs[li] * x
            for li in range(4):
                accs[32 + li] = accs[32 + li] + fs[li]
            return tuple(accs)

        accs = lax.fori_loop(0, NVEC, vec_body, accs)
        last = ch == (NCHUNK - 1)

        @pl.when(last)
        def _():
            boffs = b * ROWB
            for c in range(C):
                row = zvec
                for li in range(4):
                    s = _vsum(accs[li * C + c])
                    row = jnp.where(iota == li + 1, s, row)
                row_v[pl.ds(boffs + c * 16, 16)] = row
            crow = zvec
            for li in range(4):
                s = _vsum(accs[32 + li])
                crow = jnp.where(iota == li + 1, s, crow)
            row_v[pl.ds(boffs + C * 16, 16)] = crow

        return tuple(jnp.where(last, zvec, a) for a in accs)

    lax.fori_loop(0, NFLAT, flat_body, (zvec,) * 36)
    pltpu.sync_copy(row_v, out_hbm.at[_wid()])


@functools.partial(
    pl.kernel,
    out_type=jax.ShapeDtypeStruct((NW, 16), jnp.float32),
    mesh=_mesh,
    compiler_params=_params,
    scratch_types=[
        pltpu.VMEM((4, C, CHUNK), jnp.float32),
        pltpu.VMEM((4, CHUNK), jnp.int32),
        pltpu.VMEM((NW, BS * ROWB), jnp.float32),
        pltpu.VMEM((BS, C, 16), jnp.float32),
        pltpu.VMEM((BS, 16), jnp.float32),
        pltpu.VMEM((16,), jnp.float32),
        pltpu.SemaphoreType.DMA((4,)),
    ],
)
def _hinge(emb_hbm, seg_hbm, part_hbm, out_hbm, emb_v, seg_v, part_v,
           mu_v, coef_v, row_v, sem):
    wid = _wid()
    base = wid * PW
    zvec = jnp.zeros((16,), jnp.float32)
    iota = lax.iota(jnp.int32, 16)
    cvecs = [jnp.full((16,), c, jnp.int32) for c in range(C)]
    lmask = (iota >= 1) & (iota < L)

    pltpu.sync_copy(part_hbm, part_v)

    dsq_rows = [zvec, zvec, zvec]
    w_rows = [zvec, zvec, zvec]
    for b in range(BS):
        def red_body(w, vs):
            vs = list(vs)
            for c in range(C + 1):
                vs[c] = vs[c] + part_v[w, pl.ds(b * ROWB + c * 16, 16)]
            return tuple(vs)

        red = lax.fori_loop(0, NW, red_body, (zvec,) * (C + 1))
        vcnt = red[C]
        invv = 1.0 / jnp.maximum(vcnt, 1.0)
        presf = jnp.where(lmask & (vcnt > 0.0), 1.0, 0.0)
        nl = _vsum(presf)
        inv_nl = _srecip(jnp.maximum(nl, 1.0))
        coef_v[b, pl.ds(0, 16)] = presf * invv * inv_nl * (1.0 / BS)
        mus = [red[c] * invv for c in range(C)]
        for c in range(C):
            mu_v[b, c, pl.ds(0, 16)] = mus[c]
        guard = jnp.where(nl > 1.0, 1.0, 0.0)
        invd = guard * _srecip(jnp.maximum(nl * (nl - 1.0), 1.0)) * (1.0 / BS)
        for p, (i, j) in enumerate(_PAIRS):
            dsq = jnp.float32(0.0)
            for c in range(C):
                d = mus[c][i] - mus[c][j]
                dsq = dsq + d * d
            e = b * 6 + p
            dsq_rows[e // 16] = jnp.where(iota == e % 16, dsq, dsq_rows[e // 16])
            wv = presf[i] * presf[j] * invd
            w_rows[e // 16] = jnp.where(iota == e % 16, wv, w_rows[e // 16])

    dist = jnp.float32(0.0)
    for k in range(3):
        dsq = jnp.maximum(dsq_rows[k], 1e-24)
        pd = dsq * _vrsqrt(dsq)
        h = jnp.maximum(DELTA_D - pd, 0.0)
        dist = dist + _vsum(w_rows[k] * h * h)

    for pk in range(3):
        _issue(emb_hbm, seg_hbm, emb_v, seg_v, sem, base, pk)

    def flat_body(kk, vacc):
        par = kk & 3
        nk = kk + 3

        @pl.when(nk < NFLAT)
        def _():
            _issue(emb_hbm, seg_hbm, emb_v, seg_v, sem, base, nk)

        _drain(emb_hbm, seg_hbm, emb_v, seg_v, sem, par)
        b = lax.shift_right_logical(kk, CHSHIFT)
        bvec = jnp.full((16,), 0, jnp.int32) + b

        def vec_body(i, vacc):
            off = i * 16
            seg = seg_v[par, pl.ds(off, 16)]
            sq = jnp.zeros((16,), jnp.float32)
            for c in range(C):
                x = emb_v[par, c, pl.ds(off, 16)]
                mu = plsc.load_gather(mu_v, [bvec, cvecs[c], seg])
                d = x - mu
                sq = sq + d * d
            norm = sq * _vrsqrt(sq)
            t = jnp.maximum(norm - DELTA_V, 0.0)
            cpx = plsc.load_gather(coef_v, [bvec, seg])
            return vacc + cpx * (t * t)

        return lax.fori_loop(0, NVEC, vec_body, vacc, unroll=2)

    vacc = lax.fori_loop(0, NFLAT, flat_body, zvec)
    var_w = _vsum(vacc)

    row = (jnp.where(iota == 0, var_w, 0.0)
           + jnp.where(iota == 1, dist, 0.0)).astype(jnp.float32)
    row_v[...] = row
    pltpu.sync_copy(row_v, out_hbm.at[wid])


def kernel(embedding, seg_gt):
    partials = _moments(embedding, seg_gt)
    out = _hinge(embedding, seg_gt, partials)
    var_loss = jnp.sum(out[:, 0])
    dist_loss = out[0, 1]
    return (var_loss, dist_loss, jnp.zeros((), jnp.float32))
